# baseline (speedup 1.0000x reference)
import functools

import jax
import jax.numpy as jnp
from jax import lax
from jax.experimental import pallas as pl
from jax.experimental.pallas import tpu as pltpu
from jax.experimental.pallas import tpu_sc as plsc

B, C, D, H, W = 4, 64, 32, 32, 32
DHW = D * H * W
N = 50000

NC, NS, L = 2, 16, 16
NW = NC * NS
TILES_PER_B = NW // B
CHUNK = 128
NCHUNKS = -(-N // (TILES_PER_B * CHUNK))
PTS_PER_TILE = NCHUNKS * CHUNK
NPAD = TILES_PER_B * PTS_PER_TILE
PARTIAL = N % CHUNK

TBLK = 1024


def _tc_transpose_body(in_ref, out_ref):
    out_ref[...] = in_ref[0].T


def _build_table(feats):
    grid = (B, DHW // TBLK)
    return pl.pallas_call(
        _tc_transpose_body,
        grid=grid,
        in_specs=[pl.BlockSpec((1, C, TBLK), lambda b, k: (b, 0, k))],
        out_specs=pl.BlockSpec((TBLK, C), lambda b, k: (b * (DHW // TBLK) + k, 0)),
        out_shape=jax.ShapeDtypeStruct((B * DHW, C), jnp.float32),
    )(feats)


def _sc_body(table_hbm, grid_hbm, out_hbm, gbuf, idxbuf, rows, trans, sem):
    wid = lax.axis_index("s") * NC + lax.axis_index("c")
    b = wid // TILES_PER_B
    nbase = (wid % TILES_PER_B) * PTS_PER_TILE
    row_base = b * DHW

    def chunk_body(i, carry):
        n0 = nbase + i * CHUNK
        pltpu.sync_copy(grid_hbm.at[b, pl.ds(n0, CHUNK), :], gbuf)
        for g in range(CHUNK // L):
            pidx = lax.iota(jnp.int32, L) + g * L
            dv = plsc.load_gather(gbuf, [pidx, jnp.full((L,), 0, jnp.int32)])
            hv = plsc.load_gather(gbuf, [pidx, jnp.full((L,), 1, jnp.int32)])
            wv = plsc.load_gather(gbuf, [pidx, jnp.full((L,), 2, jnp.int32)])
            di = jnp.clip(dv.astype(jnp.int32), 0, D - 1)
            hi = jnp.clip(hv.astype(jnp.int32), 0, H - 1)
            wi = jnp.clip(wv.astype(jnp.int32), 0, W - 1)
            idxbuf[pl.ds(g * L, L)] = row_base + di * (H * W) + hi * W + wi
        pltpu.async_copy(table_hbm.at[idxbuf], rows, sem).wait()

        def c_body(c, carry2):
            cvec = jnp.full((L,), 0, jnp.int32) + c
            for j in range(CHUNK // L):
                ridx = lax.iota(jnp.int32, L) + j * L
                trans[c, pl.ds(j * L, L)] = plsc.load_gather(rows, [ridx, cvec])
            return carry2
        lax.fori_loop(0, C, c_body, 0)

        rem = N - n0
        @pl.when(rem >= CHUNK)
        def _full():
            pltpu.sync_copy(trans, out_hbm.at[b, :, pl.ds(n0, CHUNK)])

        @pl.when((rem > 0) & (rem < CHUNK))
        def _edge():
            pltpu.sync_copy(trans.at[:, pl.ds(0, PARTIAL)],
                            out_hbm.at[b, :, pl.ds(n0, PARTIAL)])
        return carry
    lax.fori_loop(0, NCHUNKS, chunk_body, 0)


_sc_gather = functools.partial(
    pl.kernel,
    out_type=jax.ShapeDtypeStruct((B, C, N), jnp.float32),
    mesh=plsc.VectorSubcoreMesh(core_axis_name="c", subcore_axis_name="s"),
    scratch_types=[
        pltpu.VMEM((CHUNK, 3), jnp.float32),
        pltpu.VMEM((CHUNK,), jnp.int32),
        pltpu.VMEM((CHUNK, C), jnp.float32),
        pltpu.VMEM((C, CHUNK), jnp.float32),
        pltpu.SemaphoreType.DMA,
    ],
    compiler_params=pltpu.CompilerParams(use_tc_tiling_on_sc=False,
                                         needs_layout_passes=False),
)(_sc_body)


def kernel(input_feats, sampling_grid):
    assert input_feats.shape == (B, C, D, H, W), input_feats.shape
    assert sampling_grid.shape == (B, N, 3), sampling_grid.shape
    table = _build_table(input_feats.reshape(B, C, DHW))
    grid_pad = jnp.pad(sampling_grid, ((0, 0), (0, NPAD - N), (0, 0)))
    return _sc_gather(table, grid_pad)


# speedup vs baseline: 1.0684x; 1.0684x over previous
import functools

import jax
import jax.numpy as jnp
from jax import lax
from jax.experimental import pallas as pl
from jax.experimental.pallas import tpu as pltpu
from jax.experimental.pallas import tpu_sc as plsc

B, C, D, H, W = 4, 64, 32, 32, 32
DHW = D * H * W
N = 50000

NC, NS, L = 2, 16, 16
NW = NC * NS
TILES_PER_B = NW // B
CHUNK = 128
NCHUNKS = 49
PTS = NCHUNKS * CHUNK
STRIDE = 6248
LAST_START = N - PTS
NGROUPS = PTS // L

TBLK = 1024


def _tc_transpose_body(in_ref, out_ref):
    out_ref[...] = in_ref[0].T


def _build_table(feats):
    grid = (B, DHW // TBLK)
    return pl.pallas_call(
        _tc_transpose_body,
        grid=grid,
        in_specs=[pl.BlockSpec((1, C, TBLK), lambda b, k: (b, 0, k))],
        out_specs=pl.BlockSpec((TBLK, C), lambda b, k: (b * (DHW // TBLK) + k, 0)),
        out_shape=jax.ShapeDtypeStruct((B * DHW, C), jnp.float32),
    )(feats)


def _transpose_chunk(rows, trans):
    ridx = [lax.iota(jnp.int32, L) + j * L for j in range(CHUNK // L)]
    for c in range(C):
        cvec = jnp.full((L,), c, jnp.int32)
        for j in range(CHUNK // L):
            trans[c, pl.ds(j * L, L)] = plsc.load_gather(rows, [ridx[j], cvec])


def _sc_body(table_hbm, grid_hbm, out_hbm,
             gbuf, idxbuf, rows0, rows1, trans0, trans1,
             sem_g0, sem_g1, sem_o0, sem_o1):
    wid = lax.axis_index("s") * NC + lax.axis_index("c")
    b = wid // TILES_PER_B
    r = wid % TILES_PER_B
    start = jnp.minimum(r * STRIDE, LAST_START)
    row_base = b * DHW

    pltpu.sync_copy(grid_hbm.at[b, pl.ds(start, PTS), :], gbuf)

    def idx_body(k, carry):
        for g in range(CHUNK // L):
            pidx = lax.iota(jnp.int32, L) + (k * CHUNK + g * L)
            dv = plsc.load_gather(gbuf, [pidx, jnp.full((L,), 0, jnp.int32)])
            hv = plsc.load_gather(gbuf, [pidx, jnp.full((L,), 1, jnp.int32)])
            wv = plsc.load_gather(gbuf, [pidx, jnp.full((L,), 2, jnp.int32)])
            di = jnp.clip(dv.astype(jnp.int32), 0, D - 1)
            hi = jnp.clip(hv.astype(jnp.int32), 0, H - 1)
            wi = jnp.clip(wv.astype(jnp.int32), 0, W - 1)
            idxbuf[pl.ds(k * CHUNK + g * L, L)] = (
                row_base + di * (H * W) + hi * W + wi)
        return carry
    lax.fori_loop(0, NCHUNKS, idx_body, 0)

    def fire_gather(k, rows, sem):
        pltpu.async_copy(table_hbm.at[idxbuf.at[pl.ds(k * CHUNK, CHUNK)]],
                         rows, sem)

    def wait_gather(rows, sem):
        pltpu.make_async_copy(table_hbm.at[idxbuf.at[pl.ds(0, CHUNK)]],
                              rows, sem).wait()

    def fire_write(k, trans, sem):
        n0 = start + k * CHUNK
        pltpu.async_copy(trans, out_hbm.at[b, :, pl.ds(n0, CHUNK)], sem)

    def wait_write(trans, sem):
        pltpu.make_async_copy(trans, out_hbm.at[b, :, pl.ds(0, CHUNK)],
                              sem).wait()

    fire_gather(0, rows0, sem_g0)

    def pipe_body(t, carry):
        k0 = 2 * t
        fire_gather(k0 + 1, rows1, sem_g1)
        wait_gather(rows0, sem_g0)

        @pl.when(t > 0)
        def _():
            wait_write(trans0, sem_o0)
        _transpose_chunk(rows0, trans0)
        fire_write(k0, trans0, sem_o0)

        fire_gather(k0 + 2, rows0, sem_g0)
        wait_gather(rows1, sem_g1)

        @pl.when(t > 0)
        def _():
            wait_write(trans1, sem_o1)
        _transpose_chunk(rows1, trans1)
        fire_write(k0 + 1, trans1, sem_o1)
        return carry
    lax.fori_loop(0, (NCHUNKS - 1) // 2, pipe_body, 0)

    wait_gather(rows0, sem_g0)
    wait_write(trans0, sem_o0)
    _transpose_chunk(rows0, trans0)
    fire_write(NCHUNKS - 1, trans0, sem_o0)
    wait_write(trans0, sem_o0)
    wait_write(trans1, sem_o1)


_sc_gather = functools.partial(
    pl.kernel,
    out_type=jax.ShapeDtypeStruct((B, C, N), jnp.float32),
    mesh=plsc.VectorSubcoreMesh(core_axis_name="c", subcore_axis_name="s"),
    scratch_types=[
        pltpu.VMEM((PTS, 3), jnp.float32),
        pltpu.VMEM((PTS,), jnp.int32),
        pltpu.VMEM((CHUNK, C), jnp.float32),
        pltpu.VMEM((CHUNK, C), jnp.float32),
        pltpu.VMEM((C, CHUNK), jnp.float32),
        pltpu.VMEM((C, CHUNK), jnp.float32),
        pltpu.SemaphoreType.DMA,
        pltpu.SemaphoreType.DMA,
        pltpu.SemaphoreType.DMA,
        pltpu.SemaphoreType.DMA,
    ],
    compiler_params=pltpu.CompilerParams(use_tc_tiling_on_sc=False,
                                         needs_layout_passes=False),
)(_sc_body)


def kernel(input_feats, sampling_grid):
    assert input_feats.shape == (B, C, D, H, W), input_feats.shape
    assert sampling_grid.shape == (B, N, 3), sampling_grid.shape
    table = _build_table(input_feats.reshape(B, C, DHW))
    return _sc_gather(table, sampling_grid)


# speedup vs baseline: 2.1568x; 2.0186x over previous
import functools

import jax
import jax.numpy as jnp
from jax import lax
from jax.experimental import pallas as pl
from jax.experimental.pallas import tpu as pltpu
from jax.experimental.pallas import tpu_sc as plsc

B, C, D, H, W = 4, 64, 32, 32, 32
DHW = D * H * W
N = 50000
TROWS = B * DHW // 2

NC, NS, L = 2, 16, 16
NW = NC * NS
TILES_PER_B = NW // B
CHUNK = 128
NCHUNKS = 49
PTS = NCHUNKS * CHUNK
XPTS = PTS + CHUNK
STRIDE = PTS
LAST_START = 43648
NG = CHUNK // L


CB = 8


def _transpose_chunk(rows, trans, offbuf, base):
    ridx = [lax.iota(jnp.int32, L) + j * L for j in range(NG)]
    offs = [offbuf[pl.ds(base + j * L, L)] for j in range(NG)]

    def cb_body(cb, carry):
        c0 = cb * CB
        for cc in range(CB):
            for j in range(NG):
                col = offs[j] + (c0 + cc)
                trans[c0 + cc, pl.ds(j * L, L)] = plsc.load_gather(
                    rows, [ridx[j], col])
        return carry
    lax.fori_loop(0, C // CB, cb_body, 0)


def _sc_body(table_hbm, grid_hbm, out_hbm,
             gbuf, idxbuf, offbuf, rows0, rows1, trans0, trans1,
             sem_g0, sem_g1, sem_o0, sem_o1):
    wid = lax.axis_index("s") * NC + lax.axis_index("c")
    b = wid // TILES_PER_B
    r = wid % TILES_PER_B
    start = jnp.minimum(r * STRIDE, LAST_START)
    row_base = b * DHW

    for coord, scale in ((0, H * W), (1, W), (2, 1)):
        pltpu.sync_copy(grid_hbm.at[coord, :, pl.ds(start, XPTS)], gbuf)

        def cpass(k, carry, coord=coord, scale=scale):
            for g in range(NG):
                p0 = k * CHUNK + g * L
                cv = gbuf[b, pl.ds(p0, L)]
                ci = jnp.clip(cv.astype(jnp.int32), 0, D - 1) * scale
                if coord == 0:
                    idxbuf[pl.ds(p0, L)] = ci + row_base
                elif coord == 1:
                    idxbuf[pl.ds(p0, L)] = idxbuf[pl.ds(p0, L)] + ci
                else:
                    v = idxbuf[pl.ds(p0, L)] + ci
                    idxbuf[pl.ds(p0, L)] = lax.shift_right_logical(v, 1)
                    offbuf[pl.ds(p0, L)] = lax.shift_left(v & 1, 6)
            return carry
        lax.fori_loop(0, NCHUNKS + 1, cpass, 0)

    def fire_gather(k, rows, sem):
        pltpu.async_copy(table_hbm.at[idxbuf.at[pl.ds(k * CHUNK, CHUNK)]],
                         rows, sem)

    def wait_gather(rows, sem):
        pltpu.make_async_copy(table_hbm.at[idxbuf.at[pl.ds(0, CHUNK)]],
                              rows, sem).wait()

    def fire_write(k, trans, sem):
        n0 = start + k * CHUNK
        pltpu.async_copy(trans, out_hbm.at[b, :, pl.ds(n0, CHUNK)], sem)

    def wait_write(trans, sem):
        pltpu.make_async_copy(trans, out_hbm.at[b, :, pl.ds(0, CHUNK)],
                              sem).wait()

    fire_gather(0, rows0, sem_g0)

    def pipe_body(t, carry):
        k0 = 2 * t
        fire_gather(k0 + 1, rows1, sem_g1)
        wait_gather(rows0, sem_g0)

        @pl.when(t > 0)
        def _():
            wait_write(trans0, sem_o0)
        _transpose_chunk(rows0, trans0, offbuf, k0 * CHUNK)
        fire_write(k0, trans0, sem_o0)

        fire_gather(k0 + 2, rows0, sem_g0)
        wait_gather(rows1, sem_g1)

        @pl.when(t > 0)
        def _():
            wait_write(trans1, sem_o1)
        _transpose_chunk(rows1, trans1, offbuf, (k0 + 1) * CHUNK)
        fire_write(k0 + 1, trans1, sem_o1)
        return carry
    lax.fori_loop(0, (NCHUNKS - 1) // 2, pipe_body, 0)

    wait_gather(rows0, sem_g0)
    wait_write(trans0, sem_o0)
    _transpose_chunk(rows0, trans0, offbuf, (NCHUNKS - 1) * CHUNK)
    fire_write(NCHUNKS - 1, trans0, sem_o0)

    @pl.when(r == TILES_PER_B - 1)
    def _():
        fire_gather(NCHUNKS, rows1, sem_g1)
        wait_gather(rows1, sem_g1)
        wait_write(trans1, sem_o1)
        _transpose_chunk(rows1, trans1, offbuf, NCHUNKS * CHUNK)
        fire_write(NCHUNKS, trans1, sem_o1)

    wait_write(trans0, sem_o0)
    wait_write(trans1, sem_o1)


_sc_gather = functools.partial(
    pl.kernel,
    out_type=jax.ShapeDtypeStruct((B, C, N), jnp.float32),
    mesh=plsc.VectorSubcoreMesh(core_axis_name="c", subcore_axis_name="s"),
    scratch_types=[
        pltpu.VMEM((B, XPTS), jnp.float32),
        pltpu.VMEM((XPTS,), jnp.int32),
        pltpu.VMEM((XPTS,), jnp.int32),
        pltpu.VMEM((CHUNK, 2 * C), jnp.float32),
        pltpu.VMEM((CHUNK, 2 * C), jnp.float32),
        pltpu.VMEM((C, CHUNK), jnp.float32),
        pltpu.VMEM((C, CHUNK), jnp.float32),
        pltpu.SemaphoreType.DMA,
        pltpu.SemaphoreType.DMA,
        pltpu.SemaphoreType.DMA,
        pltpu.SemaphoreType.DMA,
    ],
    compiler_params=pltpu.CompilerParams(use_tc_tiling_on_sc=True,
                                         needs_layout_passes=False,
                                         disable_bounds_checks=True),
)(_sc_body)


def kernel(input_feats, sampling_grid):
    assert input_feats.shape == (B, C, D, H, W), input_feats.shape
    assert sampling_grid.shape == (B, N, 3), sampling_grid.shape
    table = input_feats.transpose(0, 2, 3, 4, 1).reshape(TROWS, 2 * C)
    planes = sampling_grid.transpose(2, 0, 1)
    return _sc_gather(table, planes)
